# Initial kernel scaffold; baseline (speedup 1.0000x reference)
import functools

import jax
import jax.numpy as jnp
from jax import lax
from jax.experimental import pallas as pl
from jax.experimental.pallas import tpu as pltpu
from jax.experimental.pallas import tpu_sc as plsc

B = 100000
D = 128
S = 10
E = 128

NC, NS = 2, 16
NW = NC * NS
BP = 100352
BPW = BP // NW
C = 32
NCHUNK = BPW // C

TC_TILE = 2048


def _sc_body(feat_hbm, idx_hbm, out_hbm, idx_v, nbuf, comb, gsem, osem):
    wid = lax.axis_index("s") * NC + lax.axis_index("c")
    base = wid * BPW

    def chunk(c, _):
        cb = base + c * C
        pltpu.sync_copy(idx_hbm.at[:, pl.ds(cb, C)], idx_v)
        for s in range(S + 1):
            pltpu.async_copy(feat_hbm.at[idx_v.at[s]], nbuf.at[s], gsem)
        for s in range(S + 1):
            pltpu.make_async_copy(feat_hbm.at[idx_v.at[s]], nbuf.at[s], gsem).wait()

        def node(i, _):
            for j in range(D // 16):
                sl = pl.ds(16 * j, 16)
                comb[i, sl] = nbuf[0, i, sl]
                acc = nbuf[1, i, sl]
                for s in range(2, S + 1):
                    acc = acc + nbuf[s, i, sl]
                comb[i, pl.ds(D + 16 * j, 16)] = acc * (1.0 / S)
            return 0

        lax.fori_loop(0, C, node, 0)
        pltpu.async_copy(comb, out_hbm.at[pl.ds(cb, C)], osem)
        pltpu.make_async_copy(comb, out_hbm.at[pl.ds(cb, C)], osem).wait()
        return 0

    lax.fori_loop(0, NCHUNK, chunk, 0)


@functools.partial(
    pl.kernel,
    out_type=jax.ShapeDtypeStruct((BP, 2 * D), jnp.float32),
    mesh=plsc.VectorSubcoreMesh(
        core_axis_name="c", subcore_axis_name="s", num_cores=NC, num_subcores=NS
    ),
    scratch_types=[
        pltpu.VMEM((S + 1, C), jnp.int32),
        pltpu.VMEM((S + 1, C, D), jnp.float32),
        pltpu.VMEM((C, 2 * D), jnp.float32),
        pltpu.SemaphoreType.DMA,
        pltpu.SemaphoreType.DMA,
    ],
)
def _sc_gather(feat_hbm, idx_hbm, out_hbm, idx_v, nbuf, comb, gsem, osem):
    _sc_body(feat_hbm, idx_hbm, out_hbm, idx_v, nbuf, comb, gsem, osem)


def _mm_body(comb_ref, w_ref, out_ref):
    acc = lax.dot_general(
        w_ref[...], comb_ref[...],
        dimension_numbers=(((1,), (1,)), ((), ())),
        preferred_element_type=jnp.float32,
    )
    out_ref[...] = jnp.maximum(acc, 0.0)


def _tc_matmul(combined, w):
    return pl.pallas_call(
        _mm_body,
        grid=(BP // TC_TILE,),
        in_specs=[
            pl.BlockSpec((TC_TILE, 2 * D), lambda i: (i, 0)),
            pl.BlockSpec((E, 2 * D), lambda i: (0, 0)),
        ],
        out_specs=pl.BlockSpec((E, TC_TILE), lambda i: (0, i)),
        out_shape=jax.ShapeDtypeStruct((E, BP), jnp.float32),
    )(combined, w)


def kernel(nodes, neigh_idx, features, W):
    nodes = nodes.astype(jnp.int32)
    neigh = neigh_idx.astype(jnp.int32)
    idx_all = jnp.concatenate([nodes[None, :], neigh.T], axis=0)
    idx_all = jnp.pad(idx_all, ((0, 0), (0, BP - B)))
    combined = _sc_gather(features, idx_all)
    out = _tc_matmul(combined, W)
    return out[:, :B]


# speedup vs baseline: 2.8801x; 2.8801x over previous
import functools

import jax
import jax.numpy as jnp
from jax import lax
from jax.experimental import pallas as pl
from jax.experimental.pallas import tpu as pltpu
from jax.experimental.pallas import tpu_sc as plsc

B = 100000
D = 128
S = 10
E = 128

NC, NS = 2, 16
NW = NC * NS
BP = 100352
BPW = BP // NW
C = 32
NCHUNK = BPW // C

TC_TILE = 2048


def _sc_body(feat_hbm, idx_hbm, out_hbm, idx_v, nbuf, comb, gsem, osem):
    wid = lax.axis_index("s") * NC + lax.axis_index("c")
    base = wid * BPW

    def chunk(c, _):
        cb = base + c * C
        ib = (base // C + c) * (S + 1) * C
        pltpu.sync_copy(idx_hbm.at[pl.ds(ib, (S + 1) * C)], idx_v)
        for s in range(S + 1):
            pltpu.async_copy(feat_hbm.at[idx_v.at[pl.ds(s * C, C)]], nbuf.at[s], gsem)
        for s in range(S + 1):
            pltpu.make_async_copy(
                feat_hbm.at[idx_v.at[pl.ds(s * C, C)]], nbuf.at[s], gsem).wait()

        def node(i, _):
            for j in range(D // 16):
                sl = pl.ds(16 * j, 16)
                comb[i, sl] = nbuf[0, i, sl]
                acc = nbuf[1, i, sl]
                for s in range(2, S + 1):
                    acc = acc + nbuf[s, i, sl]
                comb[i, pl.ds(D + 16 * j, 16)] = acc * (1.0 / S)
            return 0

        lax.fori_loop(0, C, node, 0)
        pltpu.async_copy(comb, out_hbm.at[pl.ds(cb, C)], osem)
        pltpu.make_async_copy(comb, out_hbm.at[pl.ds(cb, C)], osem).wait()
        return 0

    lax.fori_loop(0, NCHUNK, chunk, 0)


@functools.partial(
    pl.kernel,
    out_type=jax.ShapeDtypeStruct((BP, 2 * D), jnp.float32),
    mesh=plsc.VectorSubcoreMesh(
        core_axis_name="c", subcore_axis_name="s", num_cores=NC, num_subcores=NS
    ),
    scratch_types=[
        pltpu.VMEM(((S + 1) * C,), jnp.int32),
        pltpu.VMEM((S + 1, C, D), jnp.float32),
        pltpu.VMEM((C, 2 * D), jnp.float32),
        pltpu.SemaphoreType.DMA,
        pltpu.SemaphoreType.DMA,
    ],
)
def _sc_gather(feat_hbm, idx_hbm, out_hbm, idx_v, nbuf, comb, gsem, osem):
    _sc_body(feat_hbm, idx_hbm, out_hbm, idx_v, nbuf, comb, gsem, osem)


def _mm_body(comb_ref, w_ref, out_ref):
    acc = lax.dot_general(
        w_ref[...], comb_ref[...],
        dimension_numbers=(((1,), (1,)), ((), ())),
        preferred_element_type=jnp.float32,
    )
    out_ref[...] = jnp.maximum(acc, 0.0)


def _tc_matmul(combined, w):
    return pl.pallas_call(
        _mm_body,
        grid=(BP // TC_TILE,),
        in_specs=[
            pl.BlockSpec((TC_TILE, 2 * D), lambda i: (i, 0)),
            pl.BlockSpec((E, 2 * D), lambda i: (0, 0)),
        ],
        out_specs=pl.BlockSpec((E, TC_TILE), lambda i: (0, i)),
        out_shape=jax.ShapeDtypeStruct((E, BP), jnp.float32),
    )(combined, w)


def kernel(nodes, neigh_idx, features, W):
    nodes = nodes.astype(jnp.int32)
    neigh = neigh_idx.astype(jnp.int32)
    idx_all = jnp.concatenate([nodes[None, :], neigh.T], axis=0)
    idx_all = jnp.pad(idx_all, ((0, 0), (0, BP - B)))
    idx_flat = (idx_all.reshape(S + 1, NW * NCHUNK, C)
                .transpose(1, 0, 2).reshape(-1))
    combined = _sc_gather(features, idx_flat)
    out = _tc_matmul(combined, W)
    return out[:, :B]


# speedup vs baseline: 3.7700x; 1.3090x over previous
import functools

import jax
import jax.numpy as jnp
from jax import lax
from jax.experimental import pallas as pl
from jax.experimental.pallas import tpu as pltpu
from jax.experimental.pallas import tpu_sc as plsc

B = 100000
D = 128
S = 10
E = 128

NC, NS = 2, 16
NW = NC * NS
BP = 100352
BPW = BP // NW
C = 32
NCHUNK = BPW // C

TC_TILE = 2048


def _sc_body(feat_hbm, idx_hbm, out_hbm, idx0, idx1, nbuf0, nbuf1,
             comb0, comb1, gsem0, gsem1, osem0, osem1):
    wid = lax.axis_index("s") * NC + lax.axis_index("c")
    base = wid * BPW
    idxs = (idx0, idx1)
    nbufs = (nbuf0, nbuf1)
    combs = (comb0, comb1)
    gsems = (gsem0, gsem1)
    osems = (osem0, osem1)

    def fire(c, b):
        ib = (base // C + c) * (S + 1) * C
        pltpu.sync_copy(idx_hbm.at[pl.ds(ib, (S + 1) * C)], idxs[b])
        for s in range(S + 1):
            pltpu.async_copy(
                feat_hbm.at[idxs[b].at[pl.ds(s * C, C)]], nbufs[b].at[s], gsems[b])

    def drain(b):
        for s in range(S + 1):
            pltpu.make_async_copy(
                feat_hbm.at[idxs[b].at[pl.ds(s * C, C)]], nbufs[b].at[s],
                gsems[b]).wait()

    def compute_store(c, b, t):
        nbuf = nbufs[b]
        comb = combs[b]

        def node(i, _):
            for j in range(D // 16):
                sl = pl.ds(16 * j, 16)
                comb[i, sl] = nbuf[0, i, sl]
                acc = nbuf[1, i, sl]
                for s in range(2, S + 1):
                    acc = acc + nbuf[s, i, sl]
                comb[i, pl.ds(D + 16 * j, 16)] = acc * (1.0 / S)
            return 0

        @pl.when(t > 0)
        def _():
            pltpu.make_async_copy(
                comb, out_hbm.at[pl.ds(base, C)], osems[b]).wait()
        lax.fori_loop(0, C, node, 0)
        pltpu.async_copy(comb, out_hbm.at[pl.ds(base + c * C, C)], osems[b])

    fire(0, 0)

    def pair(t, _):
        fire(2 * t + 1, 1)
        drain(0)
        compute_store(2 * t, 0, t)

        @pl.when(t < NCHUNK // 2 - 1)
        def _():
            fire(2 * t + 2, 0)
        drain(1)
        compute_store(2 * t + 1, 1, t)
        return 0

    lax.fori_loop(0, NCHUNK // 2, pair, 0)
    for b in (0, 1):
        pltpu.make_async_copy(
            combs[b], out_hbm.at[pl.ds(base, C)], osems[b]).wait()


@functools.partial(
    pl.kernel,
    out_type=jax.ShapeDtypeStruct((BP, 2 * D), jnp.float32),
    mesh=plsc.VectorSubcoreMesh(
        core_axis_name="c", subcore_axis_name="s", num_cores=NC, num_subcores=NS
    ),
    scratch_types=[
        pltpu.VMEM(((S + 1) * C,), jnp.int32),
        pltpu.VMEM(((S + 1) * C,), jnp.int32),
        pltpu.VMEM((S + 1, C, D), jnp.float32),
        pltpu.VMEM((S + 1, C, D), jnp.float32),
        pltpu.VMEM((C, 2 * D), jnp.float32),
        pltpu.VMEM((C, 2 * D), jnp.float32),
        pltpu.SemaphoreType.DMA,
        pltpu.SemaphoreType.DMA,
        pltpu.SemaphoreType.DMA,
        pltpu.SemaphoreType.DMA,
    ],
)
def _sc_gather(feat_hbm, idx_hbm, out_hbm, idx0, idx1, nbuf0, nbuf1,
               comb0, comb1, gsem0, gsem1, osem0, osem1):
    _sc_body(feat_hbm, idx_hbm, out_hbm, idx0, idx1, nbuf0, nbuf1,
             comb0, comb1, gsem0, gsem1, osem0, osem1)


def _mm_body(comb_ref, w_ref, out_ref):
    acc = lax.dot_general(
        w_ref[...], comb_ref[...],
        dimension_numbers=(((1,), (1,)), ((), ())),
        preferred_element_type=jnp.float32,
    )
    out_ref[...] = jnp.maximum(acc, 0.0)


def _tc_matmul(combined, w):
    return pl.pallas_call(
        _mm_body,
        grid=(BP // TC_TILE,),
        in_specs=[
            pl.BlockSpec((TC_TILE, 2 * D), lambda i: (i, 0)),
            pl.BlockSpec((E, 2 * D), lambda i: (0, 0)),
        ],
        out_specs=pl.BlockSpec((E, TC_TILE), lambda i: (0, i)),
        out_shape=jax.ShapeDtypeStruct((E, BP), jnp.float32),
    )(combined, w)


def kernel(nodes, neigh_idx, features, W):
    nodes = nodes.astype(jnp.int32)
    neigh = neigh_idx.astype(jnp.int32)
    idx_all = jnp.concatenate([nodes[None, :], neigh.T], axis=0)
    idx_all = jnp.pad(idx_all, ((0, 0), (0, BP - B)))
    idx_flat = (idx_all.reshape(S + 1, NW * NCHUNK, C)
                .transpose(1, 0, 2).reshape(-1))
    combined = _sc_gather(features, idx_flat)
    out = _tc_matmul(combined, W)
    return out[:, :B]


# speedup vs baseline: 5.2388x; 1.3896x over previous
import functools

import jax
import jax.numpy as jnp
from jax import lax
from jax.experimental import pallas as pl
from jax.experimental.pallas import tpu as pltpu
from jax.experimental.pallas import tpu_sc as plsc

B = 100000
D = 128
S = 10
E = 128

NC, NS = 2, 16
NW = NC * NS
BP = 100352
BPW = BP // NW
C = 32
NCHUNK = BPW // C

TC_TILE = 2048


NR = S * C
_STREAMS = [(0, 128), (128, 128), (256, 64)]


def _sc_body(feat_hbm, idx_hbm, out_hbm, idx0, idx1, sbuf0, sbuf1,
             nbuf0, nbuf1, mbuf0, mbuf1,
             gsem0, gsem1, ssem0, ssem1, msem0, msem1):
    wid = lax.axis_index("s") * NC + lax.axis_index("c")
    base = wid * BPW
    idxs = (idx0, idx1)
    sbufs = (sbuf0, sbuf1)
    nbufs = (nbuf0, nbuf1)
    mbufs = (mbuf0, mbuf1)
    gsems = (gsem0, gsem1)
    ssems = (ssem0, ssem1)
    msems = (msem0, msem1)

    def self_store_wait(b):
        pltpu.make_async_copy(
            sbufs[b], out_hbm.at[pl.ds(base, C), pl.ds(0, D)], ssems[b]).wait()

    def mean_store_wait(b):
        pltpu.make_async_copy(
            mbufs[b], out_hbm.at[pl.ds(base, C), pl.ds(D, D)], msems[b]).wait()

    def fire(c, b, w):
        ib = (base // C + c) * (S + 1) * C
        pltpu.sync_copy(idx_hbm.at[pl.ds(ib, (S + 1) * C)], idxs[b])

        @pl.when(w)
        def _():
            self_store_wait(b)
        pltpu.async_copy(feat_hbm.at[idxs[b].at[pl.ds(0, C)]], sbufs[b], gsems[b])
        for o, n in _STREAMS:
            pltpu.async_copy(
                feat_hbm.at[idxs[b].at[pl.ds(C + o, n)]],
                nbufs[b].at[pl.ds(o, n)], gsems[b])

    def drain(b):
        pltpu.make_async_copy(
            feat_hbm.at[idxs[b].at[pl.ds(0, C)]], sbufs[b], gsems[b]).wait()
        for o, n in _STREAMS:
            pltpu.make_async_copy(
                feat_hbm.at[idxs[b].at[pl.ds(C + o, n)]],
                nbufs[b].at[pl.ds(o, n)], gsems[b]).wait()

    def self_store(c, b):
        pltpu.async_copy(
            sbufs[b], out_hbm.at[pl.ds(base + c * C, C), pl.ds(0, D)], ssems[b])

    def compute_store(c, b, w):
        nbuf = nbufs[b]
        mbuf = mbufs[b]

        @pl.when(w)
        def _():
            mean_store_wait(b)

        def node(i, _):
            for j in range(D // 16):
                sl = pl.ds(16 * j, 16)
                acc = nbuf[i, sl]
                for s in range(1, S):
                    acc = acc + nbuf[s * C + i, sl]
                mbuf[i, sl] = acc * (1.0 / S)
            return 0

        lax.fori_loop(0, C, node, 0)
        pltpu.async_copy(
            mbuf, out_hbm.at[pl.ds(base + c * C, C), pl.ds(D, D)], msems[b])

    fire(0, 0, False)

    def pair(t, _):
        fire(2 * t + 1, 1, t > 0)
        drain(0)
        self_store(2 * t, 0)
        compute_store(2 * t, 0, t > 0)

        @pl.when(t < NCHUNK // 2 - 1)
        def _():
            fire(2 * t + 2, 0, True)
        drain(1)
        self_store(2 * t + 1, 1)
        compute_store(2 * t + 1, 1, t > 0)
        return 0

    lax.fori_loop(0, NCHUNK // 2, pair, 0)
    for b in (0, 1):
        self_store_wait(b)
        mean_store_wait(b)


@functools.partial(
    pl.kernel,
    out_type=jax.ShapeDtypeStruct((BP, 2 * D), jnp.float32),
    mesh=plsc.VectorSubcoreMesh(
        core_axis_name="c", subcore_axis_name="s", num_cores=NC, num_subcores=NS
    ),
    scratch_types=[
        pltpu.VMEM(((S + 1) * C,), jnp.int32),
        pltpu.VMEM(((S + 1) * C,), jnp.int32),
        pltpu.VMEM((C, D), jnp.float32),
        pltpu.VMEM((C, D), jnp.float32),
        pltpu.VMEM((S * C, D), jnp.float32),
        pltpu.VMEM((S * C, D), jnp.float32),
        pltpu.VMEM((C, D), jnp.float32),
        pltpu.VMEM((C, D), jnp.float32),
        pltpu.SemaphoreType.DMA,
        pltpu.SemaphoreType.DMA,
        pltpu.SemaphoreType.DMA,
        pltpu.SemaphoreType.DMA,
        pltpu.SemaphoreType.DMA,
        pltpu.SemaphoreType.DMA,
    ],
)
def _sc_gather(feat_hbm, idx_hbm, out_hbm, idx0, idx1, sbuf0, sbuf1,
               nbuf0, nbuf1, mbuf0, mbuf1,
               gsem0, gsem1, ssem0, ssem1, msem0, msem1):
    _sc_body(feat_hbm, idx_hbm, out_hbm, idx0, idx1, sbuf0, sbuf1,
             nbuf0, nbuf1, mbuf0, mbuf1,
             gsem0, gsem1, ssem0, ssem1, msem0, msem1)


def _mm_body(comb_ref, w_ref, out_ref):
    acc = lax.dot_general(
        w_ref[...], comb_ref[...],
        dimension_numbers=(((1,), (1,)), ((), ())),
        preferred_element_type=jnp.float32,
    )
    out_ref[...] = jnp.maximum(acc, 0.0)


def _tc_matmul(combined, w):
    return pl.pallas_call(
        _mm_body,
        grid=(BP // TC_TILE,),
        in_specs=[
            pl.BlockSpec((TC_TILE, 2 * D), lambda i: (i, 0)),
            pl.BlockSpec((E, 2 * D), lambda i: (0, 0)),
        ],
        out_specs=pl.BlockSpec((E, TC_TILE), lambda i: (0, i)),
        out_shape=jax.ShapeDtypeStruct((E, BP), jnp.float32),
    )(combined, w)


def kernel(nodes, neigh_idx, features, W):
    nodes = nodes.astype(jnp.int32)
    neigh = neigh_idx.astype(jnp.int32)
    idx_all = jnp.concatenate([nodes[None, :], neigh.T], axis=0)
    idx_all = jnp.pad(idx_all, ((0, 0), (0, BP - B)))
    idx_flat = (idx_all.reshape(S + 1, NW * NCHUNK, C)
                .transpose(1, 0, 2).reshape(-1))
    combined = _sc_gather(features, idx_flat)
    out = _tc_matmul(combined, W)
    return out[:, :B]
